# baseline (speedup 1.0000x reference)
import functools

import jax
import jax.numpy as jnp
from jax import lax
from jax.experimental import pallas as pl
from jax.experimental.pallas import tpu as pltpu
from jax.experimental.pallas import tpu_sc as plsc

_DIMS = [119, 5, 12, 12, 10, 6, 6, 2, 2]
_TROWS = 176
_CROWS = 328
_EMB = 128
_N = 100000
_NW = 32
_CHUNK = 80
_NCHUNKS = _N // _CHUNK
_KMAX = -(-_NCHUNKS // _NW)
_KPAIRS = (_KMAX + 1) // 2

_mesh = plsc.VectorSubcoreMesh(core_axis_name="c", subcore_axis_name="s")


@functools.partial(
    pl.kernel,
    out_type=jax.ShapeDtypeStruct((_N, _EMB), jnp.float32),
    mesh=_mesh,
    scratch_types=[
        pltpu.VMEM((_TROWS, _EMB), jnp.float32),
        pltpu.VMEM((_CROWS * _EMB,), jnp.float32),
        pltpu.VMEM((_CHUNK, 16), jnp.int32),
        pltpu.VMEM((_CHUNK, 16), jnp.int32),
        pltpu.VMEM((_CHUNK, _EMB), jnp.float32),
        pltpu.VMEM((_CHUNK, _EMB), jnp.float32),
        pltpu.SemaphoreType.DMA,
        pltpu.SemaphoreType.DMA,
        pltpu.SemaphoreType.DMA,
        pltpu.SemaphoreType.DMA,
    ],
)
def _sc_encode(x_hbm, tbl_hbm, out_hbm, tbl_v, comb_v,
               xa, xb, oa, ob, sxa, sxb, soa, sob):
    wid = lax.axis_index("s") * 2 + lax.axis_index("c")
    pltpu.sync_copy(tbl_hbm, tbl_v)

    E = _EMB

    def b12(i, c):
        def b12j(j, c2):
            dst = (i * 12 + j) * E
            for g in range(0, E, 16):
                comb_v[pl.ds(dst + g, 16)] = (
                    tbl_v[119 + i, pl.ds(g, 16)] + tbl_v[124 + j, pl.ds(g, 16)])
            return c2
        return lax.fori_loop(0, 12, b12j, c)

    lax.fori_loop(0, 5, b12, 0)

    def b34(i, c):
        def b34j(j, c2):
            dst = (60 + i * 10 + j) * E
            for g in range(0, E, 16):
                comb_v[pl.ds(dst + g, 16)] = (
                    tbl_v[136 + i, pl.ds(g, 16)] + tbl_v[148 + j, pl.ds(g, 16)])
            return c2
        return lax.fori_loop(0, 10, b34j, c)

    lax.fori_loop(0, 12, b34, 0)

    for c in range(2):
        for d in range(2):
            dst = (324 + c * 2 + d) * E
            for g in range(0, E, 16):
                comb_v[pl.ds(dst + g, 16)] = (
                    tbl_v[170 + c, pl.ds(g, 16)] + tbl_v[172 + d, pl.ds(g, 16)])

    def b5678(a, c):
        def b5678b(b, c2):
            def b5678cd(cd, c3):
                dst = (180 + (a * 6 + b) * 4 + cd) * E
                s3 = (324 + cd) * E
                for g in range(0, E, 16):
                    comb_v[pl.ds(dst + g, 16)] = (
                        tbl_v[158 + a, pl.ds(g, 16)]
                        + tbl_v[164 + b, pl.ds(g, 16)]
                        + comb_v[pl.ds(s3 + g, 16)])
                return c3
            return lax.fori_loop(0, 4, b5678cd, c2)
        return lax.fori_loop(0, 6, b5678b, c)

    lax.fori_loop(0, 6, b5678, 0)

    bufs = ((xa, oa, sxa, soa), (xb, ob, sxb, sob))

    pltpu.async_copy(x_hbm.at[pl.ds(wid * _CHUNK, _CHUNK)], xa, sxa)
    pltpu.async_copy(x_hbm.at[pl.ds((wid + _NW) * _CHUNK, _CHUNK)], xb, sxb)

    def pair_body(t, carry):
        for p in range(2):
            x_v, out_v, sx, so = bufs[p]
            k = 2 * t + p
            cid = wid + _NW * k

            @pl.when(cid < _NCHUNKS)
            def _(x_v=x_v, out_v=out_v, sx=sx, so=so, k=k, cid=cid):
                base = cid * _CHUNK
                pltpu.make_async_copy(
                    x_hbm.at[pl.ds(base, _CHUNK)], x_v, sx).wait()

                @pl.when(k >= 2)
                def _():
                    pltpu.make_async_copy(
                        out_v, out_hbm.at[pl.ds(0, _CHUNK)], so).wait()

                @plsc.parallel_loop(0, _CHUNK, 1, unroll=8)
                def row_body(r):
                    xv = x_v[r]
                    x = [xv[i] for i in range(9)]
                    cl = [x[0]] + [jnp.minimum(x[i], _DIMS[i] - 1)
                                   for i in range(1, 9)]
                    a1 = (cl[1] * 12 + cl[2]) * E
                    a2 = (60 + cl[3] * 10 + cl[4]) * E
                    a3 = (180 + (cl[5] * 6 + cl[6]) * 4
                          + cl[7] * 2 + cl[8]) * E
                    for g in range(0, E, 16):
                        acc = ((tbl_v[cl[0], pl.ds(g, 16)]
                                + comb_v[pl.ds(a1 + g, 16)])
                               + (comb_v[pl.ds(a2 + g, 16)]
                                  + comb_v[pl.ds(a3 + g, 16)]))
                        out_v[r, pl.ds(g, 16)] = acc

                pltpu.async_copy(out_v, out_hbm.at[pl.ds(base, _CHUNK)], so)

                nid = cid + 2 * _NW

                @pl.when(nid < _NCHUNKS)
                def _():
                    pltpu.async_copy(
                        x_hbm.at[pl.ds(nid * _CHUNK, _CHUNK)], x_v, sx)

        return carry

    lax.fori_loop(0, _KPAIRS, pair_body, 0)

    pltpu.make_async_copy(oa, out_hbm.at[pl.ds(0, _CHUNK)], soa).wait()
    pltpu.make_async_copy(ob, out_hbm.at[pl.ds(0, _CHUNK)], sob).wait()


def kernel(x, w0, w1, w2, w3, w4, w5, w6, w7, w8):
    tbl = jnp.concatenate([w0, w1, w2, w3, w4, w5, w6, w7, w8], axis=0)
    tbl = jnp.pad(tbl, ((0, _TROWS - tbl.shape[0]), (0, 0)))
    x16 = jnp.pad(x.astype(jnp.int32), ((0, 0), (0, 7)))
    return _sc_encode(x16, tbl)


# speedup vs baseline: 1.3219x; 1.3219x over previous
import functools

import jax
import jax.numpy as jnp
from jax import lax
from jax.experimental import pallas as pl
from jax.experimental.pallas import tpu as pltpu
from jax.experimental.pallas import tpu_sc as plsc

_DIMS = [119, 5, 12, 12, 10, 6, 6, 2, 2]
_TROWS = 176
_CROWS = 328
_EMB = 128
_N = 100000
_NW = 32
_CHUNK = 80
_NCHUNKS = _N // _CHUNK
_KMAX = -(-_NCHUNKS // _NW)
_KPAIRS = (_KMAX + 1) // 2

_mesh = plsc.VectorSubcoreMesh(core_axis_name="c", subcore_axis_name="s")


@functools.partial(
    pl.kernel,
    out_type=jax.ShapeDtypeStruct((_N, _EMB), jnp.float32),
    mesh=_mesh,
    scratch_types=[
        pltpu.VMEM((_TROWS, _EMB), jnp.float32),
        pltpu.VMEM((_CROWS * _EMB,), jnp.float32),
        pltpu.VMEM((_CHUNK, 16), jnp.int32),
        pltpu.VMEM((_CHUNK, 16), jnp.int32),
        pltpu.VMEM((_CHUNK, _EMB), jnp.float32),
        pltpu.VMEM((_CHUNK, _EMB), jnp.float32),
        pltpu.SemaphoreType.DMA,
        pltpu.SemaphoreType.DMA,
        pltpu.SemaphoreType.DMA,
        pltpu.SemaphoreType.DMA,
    ],
)
def _sc_encode(x_hbm, tbl_hbm, out_hbm, tbl_v, comb_v,
               xa, xb, oa, ob, sxa, sxb, soa, sob):
    wid = lax.axis_index("s") * 2 + lax.axis_index("c")
    pltpu.sync_copy(tbl_hbm, tbl_v)

    E = _EMB

    def b12(i, c):
        def b12j(j, c2):
            dst = (i * 12 + j) * E
            for g in range(0, E, 16):
                comb_v[pl.ds(dst + g, 16)] = (
                    tbl_v[119 + i, pl.ds(g, 16)] + tbl_v[124 + j, pl.ds(g, 16)])
            return c2
        return lax.fori_loop(0, 12, b12j, c)

    lax.fori_loop(0, 5, b12, 0)

    def b34(i, c):
        def b34j(j, c2):
            dst = (60 + i * 10 + j) * E
            for g in range(0, E, 16):
                comb_v[pl.ds(dst + g, 16)] = (
                    tbl_v[136 + i, pl.ds(g, 16)] + tbl_v[148 + j, pl.ds(g, 16)])
            return c2
        return lax.fori_loop(0, 10, b34j, c)

    lax.fori_loop(0, 12, b34, 0)

    for c in range(2):
        for d in range(2):
            dst = (324 + c * 2 + d) * E
            for g in range(0, E, 16):
                comb_v[pl.ds(dst + g, 16)] = (
                    tbl_v[170 + c, pl.ds(g, 16)] + tbl_v[172 + d, pl.ds(g, 16)])

    def b5678(a, c):
        def b5678b(b, c2):
            def b5678cd(cd, c3):
                dst = (180 + (a * 6 + b) * 4 + cd) * E
                s3 = (324 + cd) * E
                for g in range(0, E, 16):
                    comb_v[pl.ds(dst + g, 16)] = (
                        tbl_v[158 + a, pl.ds(g, 16)]
                        + tbl_v[164 + b, pl.ds(g, 16)]
                        + comb_v[pl.ds(s3 + g, 16)])
                return c3
            return lax.fori_loop(0, 4, b5678cd, c2)
        return lax.fori_loop(0, 6, b5678b, c)

    lax.fori_loop(0, 6, b5678, 0)

    bufs = ((xa, oa, sxa, soa), (xb, ob, sxb, sob))

    pltpu.async_copy(x_hbm.at[pl.ds(wid * _CHUNK, _CHUNK)], xa, sxa)
    pltpu.async_copy(x_hbm.at[pl.ds((wid + _NW) * _CHUNK, _CHUNK)], xb, sxb)

    def pair_body(t, carry):
        for p in range(2):
            x_v, out_v, sx, so = bufs[p]
            k = 2 * t + p
            cid = wid + _NW * k

            @pl.when(cid < _NCHUNKS)
            def _(x_v=x_v, out_v=out_v, sx=sx, so=so, k=k, cid=cid):
                base = cid * _CHUNK
                pltpu.make_async_copy(
                    x_hbm.at[pl.ds(base, _CHUNK)], x_v, sx).wait()

                @pl.when(k >= 2)
                def _():
                    pltpu.make_async_copy(
                        out_v, out_hbm.at[pl.ds(0, _CHUNK)], so).wait()

                @plsc.parallel_loop(0, _CHUNK, 1, unroll=4)
                def row_body(r):
                    xv = x_v[r]
                    x = [xv[i] for i in range(9)]
                    cl = [x[0]] + [jnp.minimum(x[i], _DIMS[i] - 1)
                                   for i in range(1, 9)]
                    a1 = (cl[1] * 12 + cl[2]) * E
                    a2 = (60 + cl[3] * 10 + cl[4]) * E
                    a3 = (180 + (cl[5] * 6 + cl[6]) * 4
                          + cl[7] * 2 + cl[8]) * E
                    for g in range(0, E, 16):
                        acc = ((tbl_v[cl[0], pl.ds(g, 16)]
                                + comb_v[pl.ds(a1 + g, 16)])
                               + (comb_v[pl.ds(a2 + g, 16)]
                                  + comb_v[pl.ds(a3 + g, 16)]))
                        out_v[r, pl.ds(g, 16)] = acc

                pltpu.async_copy(out_v, out_hbm.at[pl.ds(base, _CHUNK)], so)

                nid = cid + 2 * _NW

                @pl.when(nid < _NCHUNKS)
                def _():
                    pltpu.async_copy(
                        x_hbm.at[pl.ds(nid * _CHUNK, _CHUNK)], x_v, sx)

        return carry

    lax.fori_loop(0, _KPAIRS, pair_body, 0)

    pltpu.make_async_copy(oa, out_hbm.at[pl.ds(0, _CHUNK)], soa).wait()
    pltpu.make_async_copy(ob, out_hbm.at[pl.ds(0, _CHUNK)], sob).wait()


def kernel(x, w0, w1, w2, w3, w4, w5, w6, w7, w8):
    tbl = jnp.concatenate([w0, w1, w2, w3, w4, w5, w6, w7, w8], axis=0)
    tbl = jnp.pad(tbl, ((0, _TROWS - tbl.shape[0]), (0, 0)))
    x16 = jnp.pad(x.astype(jnp.int32), ((0, 0), (0, 7)))
    return _sc_encode(x16, tbl)
